# baseline (speedup 1.0000x reference)
import jax
import jax.numpy as jnp
from jax import lax
from jax.experimental import pallas as pl
from jax.experimental.pallas import tpu as pltpu
from jax.experimental.pallas import tpu_sc as plsc

N = 16384
F = 64
M = 1_000_000
T = 1024
NCH = (M + T - 1) // T
LAST_ROWS = M - (NCH - 1) * T
W = 32
B = 128
CPW = NCH // W
HI = NCH - CPW * W


def _sc_body(x_hbm, idx_hbm, out_hbm, idx_v, pos_v, d_v, acc, rowbuf, posb):
    c = lax.axis_index("c")
    s = lax.axis_index("s")
    w = s * 2 + c
    lanes = lax.iota(jnp.int32, 16)
    zf = jnp.zeros((16,), jnp.float32)
    zi = jnp.zeros((16,), jnp.int32)

    pltpu.sync_copy(idx_hbm, idx_v)

    def init_acc(r, _):
        for k in range(F // 16):
            acc[r, pl.ds(k * 16, 16)] = zf
        return 0

    lax.fori_loop(0, T, init_acc, 0)

    def init_lists(i, _):
        pos_v[pl.ds(i * 16, 16)] = i * 16 + lanes
        d_v[pl.ds(i * 16, 16)] = zi
        return 0

    lax.fori_loop(0, N // 16, init_lists, 0)

    my_cnt = jnp.where(w < HI, CPW + 1, CPW).astype(jnp.int32)
    start = (w * CPW + jnp.minimum(w, HI)).astype(jnp.int32)

    def round_body(r, _):
        g = start + r

        @pl.when(r < my_cnt)
        def _():
            base = g * T

            def scan_body(i, off):
                v = idx_v[pl.ds(i * 16, 16)]
                m = lax.shift_right_logical(v, 10) == g
                psum = plsc.cumsum(jnp.where(m, 1, 0))
                tgt = jnp.maximum(off + psum - 1, 0)
                plsc.store_scatter(pos_v, [tgt], i * 16 + lanes, mask=m)
                plsc.store_scatter(d_v, [tgt], v - base, mask=m)
                return off + jnp.max(psum)

            off = lax.fori_loop(0, N // 16, scan_body, jnp.int32(0))
            nb = (off + (B - 1)) // B

            def batch_body(b, _):
                b0 = b * B
                for q in range(B // 16):
                    posb[pl.ds(q * 16, 16)] = pos_v[pl.ds(b0 + q * 16, 16)]
                pltpu.sync_copy(x_hbm.at[posb], rowbuf)
                for jg in range(B // 16):
                    jv = jg * 16 + lanes
                    dv = d_v[pl.ds(b0 + jg * 16, 16)]
                    valid = (b0 + jg * 16 + lanes) < off
                    for k in range(F):
                        kv = jnp.full((16,), k, jnp.int32)
                        val = plsc.load_gather(rowbuf, [jv, kv])
                        plsc.addupdate_scatter(acc, [dv, kv], val, mask=valid)
                return 0

            lax.fori_loop(0, nb, batch_body, 0)

            @pl.when(g != NCH - 1)
            def _():
                pltpu.sync_copy(acc, out_hbm.at[pl.ds(base, T)])

            @pl.when(g == NCH - 1)
            def _():
                pltpu.sync_copy(acc.at[pl.ds(0, LAST_ROWS)],
                                out_hbm.at[pl.ds(base, LAST_ROWS)])

            def zero_body(b, _):
                b0 = b * B
                for jg in range(B // 16):
                    dv = d_v[pl.ds(b0 + jg * 16, 16)]
                    valid = (b0 + jg * 16 + lanes) < off
                    for k in range(F):
                        kv = jnp.full((16,), k, jnp.int32)
                        plsc.store_scatter(acc, [dv, kv], zf, mask=valid)
                return 0

            lax.fori_loop(0, nb, zero_body, 0)

        return 0

    lax.fori_loop(0, CPW + 1, round_body, 0)


@jax.jit
def _impl(x_data, idx32):
    kern = pl.kernel(
        _sc_body,
        out_type=jax.ShapeDtypeStruct((M, F), jnp.float32),
        mesh=plsc.VectorSubcoreMesh(core_axis_name="c", subcore_axis_name="s"),
        compiler_params=pltpu.CompilerParams(
            needs_layout_passes=False, use_tc_tiling_on_sc=False),
        scratch_types=[
            pltpu.VMEM((N,), jnp.int32),
            pltpu.VMEM((N,), jnp.int32),
            pltpu.VMEM((N,), jnp.int32),
            pltpu.VMEM((T, F), jnp.float32),
            pltpu.VMEM((B, F), jnp.float32),
            pltpu.VMEM((B,), jnp.int32),
        ],
    )
    return kern(x_data, idx32)


def kernel(x_data, scatter_idcs, protoshape):
    idx32 = scatter_idcs[:, 0].astype(jnp.int32)
    return _impl(x_data, idx32)


# speedup vs baseline: 1.5828x; 1.5828x over previous
import jax
import jax.numpy as jnp
from jax import lax
from jax.experimental import pallas as pl
from jax.experimental.pallas import tpu as pltpu
from jax.experimental.pallas import tpu_sc as plsc

N = 16384
F = 64
M = 1_000_000
MP = 1_000_064
T = 1024
NCH = (MP + T - 1) // T
LAST_ROWS = MP - (NCH - 1) * T
W = 32
B = 64
CPW = NCH // W
HI = NCH - CPW * W


def _sc_body(x_hbm, idx_hbm, out_hbm, idx_v, pos_v, dc_v, acc, rowbuf, posb):
    c = lax.axis_index("c")
    s = lax.axis_index("s")
    w = s * 2 + c
    lanes = lax.iota(jnp.int32, 16)
    zf = jnp.zeros((16,), jnp.float32)
    zi = jnp.zeros((16,), jnp.int32)

    pltpu.sync_copy(idx_hbm, idx_v)

    def init_acc(i, _):
        for a in range(8):
            for b in range(8):
                acc[a, b, pl.ds(i * 16, 16)] = zf
        return 0

    lax.fori_loop(0, T // 16, init_acc, 0)

    def init_lists(i, _):
        pos_v[pl.ds(i * 16, 16)] = (i * 16 + lanes) >> 1
        dc_v[pl.ds(i * 16, 16)] = zi
        return 0

    lax.fori_loop(0, N // 16, init_lists, 0)

    my_cnt = jnp.where(w < HI, CPW + 1, CPW).astype(jnp.int32)
    start = (w * CPW + jnp.minimum(w, HI)).astype(jnp.int32)

    def round_body(r, _):
        g = start + r

        @pl.when(r < my_cnt)
        def _():
            base = g * T

            def scan_body(i, off):
                v = idx_v[pl.ds(i * 16, 16)]
                m = lax.shift_right_logical(v, 10) == g
                psum = plsc.cumsum(jnp.where(m, 1, 0))
                tgt = jnp.maximum(off + psum - 1, 0)
                pos = i * 16 + lanes
                plsc.store_scatter(pos_v, [tgt], pos >> 1, mask=m)
                dc = (v - base) | ((pos & 1) << 10)
                plsc.store_scatter(dc_v, [tgt], dc, mask=m)
                return off + jnp.max(psum)

            off = lax.fori_loop(0, N // 16, scan_body, jnp.int32(0))
            nb = (off + (B - 1)) // B

            def batch_body(b, _):
                b0 = b * B
                for q in range(B // 16):
                    posb[pl.ds(q * 16, 16)] = pos_v[pl.ds(b0 + q * 16, 16)]
                pltpu.sync_copy(x_hbm.at[posb], rowbuf)
                for jg in range(B // 16):
                    jv = jg * 16 + lanes
                    dcv = dc_v[pl.ds(b0 + jg * 16, 16)]
                    dv = dcv & 1023
                    colb = lax.shift_right_logical(dcv, 10) * 64
                    valid = (b0 + jg * 16 + lanes) < off
                    for k in range(F):
                        av = jnp.full((16,), k >> 3, jnp.int32)
                        bv = jnp.full((16,), k & 7, jnp.int32)
                        val = plsc.load_gather(rowbuf, [jv, colb + k])
                        plsc.addupdate_scatter(acc, [av, bv, dv], val,
                                               mask=valid)
                return 0

            lax.fori_loop(0, nb, batch_body, 0)

            @pl.when(g != NCH - 1)
            def _():
                for a in range(8):
                    pltpu.sync_copy(
                        acc.at[a],
                        out_hbm.at[pl.ds(a * 8, 8), pl.ds(base, T)])

            @pl.when(g == NCH - 1)
            def _():
                for a in range(8):
                    pltpu.sync_copy(
                        acc.at[a, slice(None), pl.ds(0, LAST_ROWS)],
                        out_hbm.at[pl.ds(a * 8, 8), pl.ds(base, LAST_ROWS)])

            def zero_body(b, _):
                b0 = b * B
                for jg in range(B // 16):
                    dcv = dc_v[pl.ds(b0 + jg * 16, 16)]
                    dv = dcv & 1023
                    valid = (b0 + jg * 16 + lanes) < off
                    for k in range(F):
                        av = jnp.full((16,), k >> 3, jnp.int32)
                        bv = jnp.full((16,), k & 7, jnp.int32)
                        plsc.store_scatter(acc, [av, bv, dv], zf, mask=valid)
                return 0

            lax.fori_loop(0, nb, zero_body, 0)

        return 0

    lax.fori_loop(0, CPW + 1, round_body, 0)


@jax.jit
def _impl(x2, idx32):
    kern = pl.kernel(
        _sc_body,
        out_type=jax.ShapeDtypeStruct((F, MP), jnp.float32),
        mesh=plsc.VectorSubcoreMesh(core_axis_name="c", subcore_axis_name="s"),
        compiler_params=pltpu.CompilerParams(needs_layout_passes=False),
        scratch_types=[
            pltpu.VMEM((N,), jnp.int32),
            pltpu.VMEM((N,), jnp.int32),
            pltpu.VMEM((N,), jnp.int32),
            pltpu.VMEM((8, 8, T), jnp.float32),
            pltpu.VMEM((B, 128), jnp.float32),
            pltpu.VMEM((B,), jnp.int32),
        ],
    )
    return kern(x2, idx32)


def kernel(x_data, scatter_idcs, protoshape):
    idx32 = scatter_idcs[:, 0].astype(jnp.int32)
    x2 = x_data.reshape(N // 2, 128)
    return _impl(x2, idx32)[:, :M].T


# speedup vs baseline: 1.9117x; 1.2078x over previous
import jax
import jax.numpy as jnp
from jax import lax
from jax.experimental import pallas as pl
from jax.experimental.pallas import tpu as pltpu
from jax.experimental.pallas import tpu_sc as plsc

N = 16384
F = 64
M = 1_000_000
MP = 1_000_064
T = 1024
NCH = (MP + T - 1) // T
LAST_ROWS = MP - (NCH - 1) * T
W = 32
B = 64
CPW = NCH // W
HI = NCH - CPW * W


def _sc_body(x_hbm, idx_hbm, out_hbm, idx_v, pos_v, dc_v, acc, rowbuf, posb):
    c = lax.axis_index("c")
    s = lax.axis_index("s")
    w = s * 2 + c
    lanes = lax.iota(jnp.int32, 16)
    zf = jnp.zeros((16,), jnp.float32)
    zi = jnp.zeros((16,), jnp.int32)

    pltpu.sync_copy(idx_hbm, idx_v)

    def init_acc(i, _):
        for a in range(8):
            for b in range(8):
                acc[a, b, pl.ds(i * 16, 16)] = zf
        return 0

    lax.fori_loop(0, T // 16, init_acc, 0)

    def init_lists(i, _):
        pos_v[pl.ds(i * 16, 16)] = (i * 16 + lanes) >> 1
        dc_v[pl.ds(i * 16, 16)] = zi
        return 0

    lax.fori_loop(0, N // 16, init_lists, 0)

    my_cnt = jnp.where(w < HI, CPW + 1, CPW).astype(jnp.int32)
    start = (w * CPW + jnp.minimum(w, HI)).astype(jnp.int32)

    def round_body(r, _):
        g = start + r

        @pl.when(r < my_cnt)
        def _():
            base = g * T

            def scan_body(i, off):
                v = idx_v[pl.ds(i * 16, 16)]
                m = lax.shift_right_logical(v, 10) == g
                psum = plsc.cumsum(jnp.where(m, 1, 0))
                tgt = jnp.maximum(off + psum - 1, 0)
                pos = i * 16 + lanes
                plsc.store_scatter(pos_v, [tgt], pos >> 1, mask=m)
                dc = (v - base) | ((pos & 1) << 10)
                plsc.store_scatter(dc_v, [tgt], dc, mask=m)
                return off + jnp.max(psum)

            off = lax.fori_loop(0, N // 16, scan_body, jnp.int32(0))
            nb = (off + (B - 1)) // B

            def batch_body(b, _):
                b0 = b * B
                for q in range(B // 16):
                    posb[pl.ds(q * 16, 16)] = pos_v[pl.ds(b0 + q * 16, 16)]
                pltpu.sync_copy(x_hbm.at[posb], rowbuf)
                for jg in range(B // 16):
                    jv = jg * 16 + lanes
                    dcv = dc_v[pl.ds(b0 + jg * 16, 16)]
                    dv = dcv & 1023
                    colb = lax.shift_right_logical(dcv, 10) * 64
                    valid = (b0 + jg * 16 + lanes) < off
                    for k in range(F):
                        av = jnp.full((16,), k >> 3, jnp.int32)
                        bv = jnp.full((16,), k & 7, jnp.int32)
                        val = plsc.load_gather(rowbuf, [jv, colb + k])
                        plsc.addupdate_scatter(acc, [av, bv, dv], val,
                                               mask=valid)
                return 0

            lax.fori_loop(0, nb, batch_body, 0)

            @pl.when(g != NCH - 1)
            def _():
                for a in range(8):
                    pltpu.sync_copy(
                        acc.at[a],
                        out_hbm.at[pl.ds(a * 8, 8), pl.ds(base, T)])

            @pl.when(g == NCH - 1)
            def _():
                for a in range(8):
                    pltpu.sync_copy(
                        acc.at[a, slice(None), pl.ds(0, LAST_ROWS)],
                        out_hbm.at[pl.ds(a * 8, 8), pl.ds(base, LAST_ROWS)])

            def zero_body(b, _):
                b0 = b * B
                for jg in range(B // 16):
                    dcv = dc_v[pl.ds(b0 + jg * 16, 16)]
                    dv = dcv & 1023
                    valid = (b0 + jg * 16 + lanes) < off
                    for k in range(F):
                        av = jnp.full((16,), k >> 3, jnp.int32)
                        bv = jnp.full((16,), k & 7, jnp.int32)
                        plsc.store_scatter(acc, [av, bv, dv], zf, mask=valid)
                return 0

            lax.fori_loop(0, nb, zero_body, 0)

        return 0

    lax.fori_loop(0, CPW + 1, round_body, 0)


@jax.jit
def _impl(x2, idx32):
    kern = pl.kernel(
        _sc_body,
        out_type=jax.ShapeDtypeStruct((F, M), jnp.float32),
        mesh=plsc.VectorSubcoreMesh(core_axis_name="c", subcore_axis_name="s"),
        compiler_params=pltpu.CompilerParams(needs_layout_passes=False),
        scratch_types=[
            pltpu.VMEM((N,), jnp.int32),
            pltpu.VMEM((N,), jnp.int32),
            pltpu.VMEM((N,), jnp.int32),
            pltpu.VMEM((8, 8, T), jnp.float32),
            pltpu.VMEM((B, 128), jnp.float32),
            pltpu.VMEM((B,), jnp.int32),
        ],
    )
    return kern(x2, idx32)


def kernel(x_data, scatter_idcs, protoshape):
    idx32 = scatter_idcs[:, 0].astype(jnp.int32)
    x2 = x_data.reshape(N // 2, 128)
    return _impl(x2, idx32).T


# speedup vs baseline: 2.0385x; 1.0663x over previous
import jax
import jax.numpy as jnp
from jax import lax
from jax.experimental import pallas as pl
from jax.experimental.pallas import tpu as pltpu
from jax.experimental.pallas import tpu_sc as plsc

N = 16384
F = 64
M = 1_000_000
MP = 1_000_064
T = 1024
NCH = (MP + T - 1) // T
LAST_ROWS = MP - (NCH - 1) * T
W = 32
B = 64
CPW = NCH // W
HI = NCH - CPW * W


def _sc_body(x_hbm, idx_hbm, out_hbm, idx_v, pos_v, dc_v, acc, rowbuf, posb):
    c = lax.axis_index("c")
    s = lax.axis_index("s")
    w = s * 2 + c
    lanes = lax.iota(jnp.int32, 16)
    zf = jnp.zeros((16,), jnp.float32)
    zi = jnp.zeros((16,), jnp.int32)

    pltpu.sync_copy(idx_hbm, idx_v)

    def init_acc(i, _):
        for a in range(8):
            for b in range(8):
                acc[a, b, pl.ds(i * 16, 16)] = zf
        return 0

    lax.fori_loop(0, T // 16, init_acc, 0)

    def init_lists(i, _):
        pos_v[pl.ds(i * 16, 16)] = (i * 16 + lanes) >> 1
        dc_v[pl.ds(i * 16, 16)] = zi
        return 0

    lax.fori_loop(0, N // 16, init_lists, 0)

    my_cnt = jnp.where(w < HI, CPW + 1, CPW).astype(jnp.int32)
    start = (w * CPW + jnp.minimum(w, HI)).astype(jnp.int32)

    def round_body(r, _):
        g = start + r

        @pl.when(r < my_cnt)
        def _():
            base = g * T

            def scan_body(i4, off):
                offs = off
                for u in range(4):
                    i = i4 * 4 + u
                    v = idx_v[pl.ds(i * 16, 16)]
                    m = lax.shift_right_logical(v, 10) == g
                    psum = plsc.cumsum(jnp.where(m, 1, 0))
                    tgt = jnp.maximum(offs + psum - 1, 0)
                    pos = i * 16 + lanes
                    plsc.store_scatter(pos_v, [tgt], pos >> 1, mask=m)
                    dc = (v - base) | ((pos & 1) << 10)
                    plsc.store_scatter(dc_v, [tgt], dc, mask=m)
                    cnt = plsc.all_reduce_population_count(m)
                    offs = offs + cnt[0]
                return offs

            off = lax.fori_loop(0, N // 64, scan_body, jnp.int32(0))
            nb = (off + (B - 1)) // B

            def batch_body(b, _):
                b0 = b * B
                for q in range(B // 16):
                    posb[pl.ds(q * 16, 16)] = pos_v[pl.ds(b0 + q * 16, 16)]
                pltpu.sync_copy(x_hbm.at[posb], rowbuf)
                for jg in range(B // 16):
                    jv = jg * 16 + lanes
                    dcv = dc_v[pl.ds(b0 + jg * 16, 16)]
                    dv = dcv & 1023
                    colb = lax.shift_right_logical(dcv, 10) * 64
                    valid = (b0 + jg * 16 + lanes) < off
                    for k in range(F):
                        av = jnp.full((16,), k >> 3, jnp.int32)
                        bv = jnp.full((16,), k & 7, jnp.int32)
                        val = plsc.load_gather(rowbuf, [jv, colb + k])
                        plsc.addupdate_scatter(acc, [av, bv, dv], val,
                                               mask=valid)
                return 0

            lax.fori_loop(0, nb, batch_body, 0)

            @pl.when(g != NCH - 1)
            def _():
                for a in range(8):
                    pltpu.sync_copy(
                        acc.at[a],
                        out_hbm.at[pl.ds(a * 8, 8), pl.ds(base, T)])

            @pl.when(g == NCH - 1)
            def _():
                for a in range(8):
                    pltpu.sync_copy(
                        acc.at[a, slice(None), pl.ds(0, LAST_ROWS)],
                        out_hbm.at[pl.ds(a * 8, 8), pl.ds(base, LAST_ROWS)])

            def zero_body(b, _):
                b0 = b * B
                for jg in range(B // 16):
                    dcv = dc_v[pl.ds(b0 + jg * 16, 16)]
                    dv = dcv & 1023
                    valid = (b0 + jg * 16 + lanes) < off
                    for k in range(F):
                        av = jnp.full((16,), k >> 3, jnp.int32)
                        bv = jnp.full((16,), k & 7, jnp.int32)
                        plsc.store_scatter(acc, [av, bv, dv], zf, mask=valid)
                return 0

            lax.fori_loop(0, nb, zero_body, 0)

        return 0

    lax.fori_loop(0, CPW + 1, round_body, 0)


@jax.jit
def _impl(x2, idx32):
    kern = pl.kernel(
        _sc_body,
        out_type=jax.ShapeDtypeStruct((F, M), jnp.float32),
        mesh=plsc.VectorSubcoreMesh(core_axis_name="c", subcore_axis_name="s"),
        compiler_params=pltpu.CompilerParams(needs_layout_passes=False),
        scratch_types=[
            pltpu.VMEM((N,), jnp.int32),
            pltpu.VMEM((N,), jnp.int32),
            pltpu.VMEM((N,), jnp.int32),
            pltpu.VMEM((8, 8, T), jnp.float32),
            pltpu.VMEM((B, 128), jnp.float32),
            pltpu.VMEM((B,), jnp.int32),
        ],
    )
    return kern(x2, idx32)


def kernel(x_data, scatter_idcs, protoshape):
    idx32 = scatter_idcs[:, 0].astype(jnp.int32)
    x2 = x_data.reshape(N // 2, 128)
    return _impl(x2, idx32).T


# speedup vs baseline: 2.4754x; 1.2144x over previous
import jax
import jax.numpy as jnp
from jax import lax
from jax.experimental import pallas as pl
from jax.experimental.pallas import tpu as pltpu
from jax.experimental.pallas import tpu_sc as plsc

N = 16384
F = 64
M = 1_000_000
MP = 1_000_064
T = 1024
NCH = (MP + T - 1) // T
LAST_ROWS = MP - (NCH - 1) * T
W = 32
B = 32
AP = T + 17
CPW = NCH // W
HI = NCH - CPW * W


def _sc_body(x_hbm, idx_hbm, out_hbm, idx_v, mine_v, rnd_v, acc, rowbuf,
             posb):
    c = lax.axis_index("c")
    s = lax.axis_index("s")
    w = s * 2 + c
    lanes = lax.iota(jnp.int32, 16)
    zf = jnp.zeros((16,), jnp.float32)

    pltpu.sync_copy(idx_hbm, idx_v)

    def init_acc(r2, _):
        for q in range(T // 16):
            acc[r2, pl.ds(q * 16, 16)] = zf
        acc[r2, pl.ds(AP - 16, 16)] = zf
        return 0

    lax.fori_loop(0, F, init_acc, 0)

    my_cnt = jnp.where(w < HI, CPW + 1, CPW).astype(jnp.int32)
    start = (w * CPW + jnp.minimum(w, HI)).astype(jnp.int32)
    lo = start * T
    hi = (start + my_cnt) * T

    def p1_body(i, moff):
        v = idx_v[pl.ds(i * 16, 16)]
        m = (v >= lo) & (v < hi)
        psum = plsc.cumsum(jnp.where(m, 1, 0))
        tgt = jnp.maximum(moff + psum - 1, 0)
        pos = i * 16 + lanes
        e = (pos >> 1) | ((pos & 1) << 13) | ((v - lo) << 14)
        plsc.store_scatter(mine_v, [tgt], e, mask=m)
        return moff + jnp.max(psum)

    mcnt = lax.fori_loop(0, N // 16, p1_body, jnp.int32(0))
    n_mv = (mcnt + 15) >> 4

    def round_body(r, _):
        g = start + r

        @pl.when(r < my_cnt)
        def _():
            base = g * T
            rlo = r * T

            def comp_body(i, roff):
                e = mine_v[pl.ds(i * 16, 16)]
                dreg = lax.shift_right_logical(e, 14)
                m = ((dreg >= rlo) & (dreg < rlo + T)
                     & ((i * 16 + lanes) < mcnt))
                psum = plsc.cumsum(jnp.where(m, 1, 0))
                tgt = jnp.maximum(roff + psum - 1, 0)
                re = (e & 0x3FFF) | ((dreg - rlo) << 14)
                plsc.store_scatter(rnd_v, [tgt], re, mask=m)
                return roff + jnp.max(psum)

            rcnt = lax.fori_loop(0, n_mv, comp_body, jnp.int32(0))
            nb = (rcnt + (B - 1)) // B

            def batch_body(b, _):
                b0 = b * B
                for q in range(B // 16):
                    e = rnd_v[pl.ds(b0 + q * 16, 16)]
                    posb[pl.ds(q * 16, 16)] = e & 0x1FFF
                pltpu.sync_copy(x_hbm.at[posb], rowbuf)
                for q in range(B // 16):
                    e_vec = rnd_v[pl.ds(b0 + q * 16, 16)]
                    for u in range(16):
                        uu = q * 16 + u
                        eb = e_vec.at[jnp.full((16,), u, jnp.int32)].get(
                            mode="promise_in_bounds")
                        colb = (lax.shift_right_logical(eb, 13) & 1) * 64
                        dv = lax.shift_right_logical(eb, 14)
                        validv = jnp.full((16,), b0 + uu, jnp.int32) < rcnt
                        usplat = jnp.full((16,), uu, jnp.int32)
                        for kq in range(F // 16):
                            kvec = kq * 16 + lanes
                            val = plsc.load_gather(rowbuf,
                                                   [usplat, colb + kvec])
                            plsc.addupdate_scatter(acc, [kvec, dv], val,
                                                   mask=validv)
                return 0

            lax.fori_loop(0, nb, batch_body, 0)

            @pl.when(g != NCH - 1)
            def _():
                pltpu.sync_copy(acc.at[:, pl.ds(0, T)],
                                out_hbm.at[:, pl.ds(base, T)])

            @pl.when(g == NCH - 1)
            def _():
                pltpu.sync_copy(acc.at[:, pl.ds(0, LAST_ROWS)],
                                out_hbm.at[:, pl.ds(base, LAST_ROWS)])

            def zero_body(b, _):
                b0 = b * B
                for q in range(B // 16):
                    e_vec = rnd_v[pl.ds(b0 + q * 16, 16)]
                    for u in range(16):
                        uu = q * 16 + u
                        eb = e_vec.at[jnp.full((16,), u, jnp.int32)].get(
                            mode="promise_in_bounds")
                        dv = lax.shift_right_logical(eb, 14)
                        validv = jnp.full((16,), b0 + uu, jnp.int32) < rcnt
                        for kq in range(F // 16):
                            kvec = kq * 16 + lanes
                            plsc.store_scatter(acc, [kvec, dv], zf,
                                               mask=validv)
                return 0

            lax.fori_loop(0, nb, zero_body, 0)

        return 0

    lax.fori_loop(0, CPW + 1, round_body, 0)


@jax.jit
def _impl(x2, idx32):
    kern = pl.kernel(
        _sc_body,
        out_type=jax.ShapeDtypeStruct((F, M), jnp.float32),
        mesh=plsc.VectorSubcoreMesh(core_axis_name="c", subcore_axis_name="s"),
        compiler_params=pltpu.CompilerParams(needs_layout_passes=False),
        scratch_types=[
            pltpu.VMEM((N,), jnp.int32),
            pltpu.VMEM((N,), jnp.int32),
            pltpu.VMEM((N,), jnp.int32),
            pltpu.VMEM((F, AP), jnp.float32),
            pltpu.VMEM((B, 128), jnp.float32),
            pltpu.VMEM((B,), jnp.int32),
        ],
    )
    return kern(x2, idx32)


def kernel(x_data, scatter_idcs, protoshape):
    idx32 = scatter_idcs[:, 0].astype(jnp.int32)
    x2 = x_data.reshape(N // 2, 128)
    return _impl(x2, idx32).T


# speedup vs baseline: 5.7386x; 2.3183x over previous
import jax
import jax.numpy as jnp
from jax import lax
from jax.experimental import pallas as pl
from jax.experimental.pallas import tpu as pltpu
from jax.experimental.pallas import tpu_sc as plsc

N = 16384
F = 64
M = 1_000_000
MP = 1_000_064
T = 1024
NCH = (MP + T - 1) // T
LAST_ROWS = MP - (NCH - 1) * T
W = 32
B = 32
CPW = NCH // W
HI = NCH - CPW * W


def _sc_body(x_hbm, idx_hbm, out_hbm, mine_v, rnd_v, acc, rowbuf,
             posb):
    c = lax.axis_index("c")
    s = lax.axis_index("s")
    w = s * 2 + c
    lanes = lax.iota(jnp.int32, 16)
    zf = jnp.zeros((16,), jnp.float32)

    pltpu.sync_copy(idx_hbm, rnd_v)

    def init_acc(i, _):
        for a in range(8):
            for b in range(8):
                acc[a, b, pl.ds(i * 16, 16)] = zf
        return 0

    lax.fori_loop(0, T // 16, init_acc, 0)

    my_cnt = jnp.where(w < HI, CPW + 1, CPW).astype(jnp.int32)
    start = (w * CPW + jnp.minimum(w, HI)).astype(jnp.int32)
    lo = start * T
    hi = (start + my_cnt) * T

    def p1_body(i, moff):
        v = rnd_v[pl.ds(i * 16, 16)]
        m = (v >= lo) & (v < hi)
        psum = plsc.cumsum(jnp.where(m, 1, 0))
        tgt = jnp.maximum(moff + psum - 1, 0)
        pos = i * 16 + lanes
        e = (pos >> 1) | ((pos & 1) << 13) | ((v - lo) << 14)
        plsc.store_scatter(mine_v, [tgt], e, mask=m)
        return moff + jnp.max(psum)

    mcnt = lax.fori_loop(0, N // 16, p1_body, jnp.int32(0))
    n_mv = (mcnt + 15) >> 4

    def round_body(r, _):
        g = start + r

        @pl.when(r < my_cnt)
        def _():
            base = g * T
            rlo = r * T

            def comp_body(i, roff):
                e = mine_v[pl.ds(i * 16, 16)]
                dreg = lax.shift_right_logical(e, 14)
                m = ((dreg >= rlo) & (dreg < rlo + T)
                     & ((i * 16 + lanes) < mcnt))
                psum = plsc.cumsum(jnp.where(m, 1, 0))
                tgt = jnp.maximum(roff + psum - 1, 0)
                re = (e & 0x3FFF) | ((dreg - rlo) << 14)
                plsc.store_scatter(rnd_v, [tgt], re, mask=m)
                return roff + jnp.max(psum)

            rcnt = lax.fori_loop(0, n_mv, comp_body, jnp.int32(0))
            nb = (rcnt + (B - 1)) // B

            def batch_body(b, _):
                b0 = b * B
                for q in range(B // 16):
                    e = rnd_v[pl.ds(b0 + q * 16, 16)]
                    posb[pl.ds(q * 16, 16)] = e & 0x1FFF
                pltpu.sync_copy(x_hbm.at[posb], rowbuf.at[:, pl.ds(0, 128)])
                for jg in range(B // 16):
                    jv = jg * 16 + lanes
                    e_vec = rnd_v[pl.ds(b0 + jg * 16, 16)]
                    dv = lax.shift_right_logical(e_vec, 14)
                    colb = (lax.shift_right_logical(e_vec, 13) & 1) * 64
                    valid = (b0 + jg * 16 + lanes) < rcnt
                    for k in range(F):
                        av = jnp.full((16,), k >> 3, jnp.int32)
                        bv = jnp.full((16,), k & 7, jnp.int32)
                        val = plsc.load_gather(rowbuf, [jv, colb + k])
                        plsc.addupdate_scatter(acc, [av, bv, dv], val,
                                               mask=valid)
                return 0

            lax.fori_loop(0, nb, batch_body, 0)

            @pl.when(g != NCH - 1)
            def _():
                for a in range(8):
                    pltpu.sync_copy(
                        acc.at[a],
                        out_hbm.at[pl.ds(a * 8, 8), pl.ds(base, T)])

            @pl.when(g == NCH - 1)
            def _():
                for a in range(8):
                    pltpu.sync_copy(
                        acc.at[a, slice(None), pl.ds(0, LAST_ROWS)],
                        out_hbm.at[pl.ds(a * 8, 8), pl.ds(base, LAST_ROWS)])

            def zero_body(b, _):
                b0 = b * B
                for jg in range(B // 16):
                    e_vec = rnd_v[pl.ds(b0 + jg * 16, 16)]
                    dv = lax.shift_right_logical(e_vec, 14)
                    valid = (b0 + jg * 16 + lanes) < rcnt
                    for k in range(F):
                        av = jnp.full((16,), k >> 3, jnp.int32)
                        bv = jnp.full((16,), k & 7, jnp.int32)
                        plsc.store_scatter(acc, [av, bv, dv], zf, mask=valid)
                return 0

            lax.fori_loop(0, nb, zero_body, 0)

        return 0

    lax.fori_loop(0, CPW + 1, round_body, 0)


@jax.jit
def _impl(x2, idx32):
    kern = pl.kernel(
        _sc_body,
        out_type=jax.ShapeDtypeStruct((F, M), jnp.float32),
        mesh=plsc.VectorSubcoreMesh(core_axis_name="c", subcore_axis_name="s"),
        compiler_params=pltpu.CompilerParams(needs_layout_passes=False),
        scratch_types=[
            pltpu.VMEM((N,), jnp.int32),
            pltpu.VMEM((N,), jnp.int32),
            pltpu.VMEM((8, 8, T), jnp.float32),
            pltpu.VMEM((B, 129), jnp.float32),
            pltpu.VMEM((B,), jnp.int32),
        ],
    )
    return kern(x2, idx32)


def kernel(x_data, scatter_idcs, protoshape):
    idx32 = scatter_idcs[:, 0].astype(jnp.int32)
    x2 = x_data.reshape(N // 2, 128)
    return _impl(x2, idx32).T


# speedup vs baseline: 5.8033x; 1.0113x over previous
import jax
import jax.numpy as jnp
from jax import lax
from jax.experimental import pallas as pl
from jax.experimental.pallas import tpu as pltpu
from jax.experimental.pallas import tpu_sc as plsc

N = 16384
F = 64
M = 1_000_000
MP = 1_000_064
T = 1024
NCH = (MP + T - 1) // T
LAST_ROWS = MP - (NCH - 1) * T
W = 32
B = 32
CPW = NCH // W
HI = NCH - CPW * W


def _sc_body(x_hbm, idx_hbm, out_hbm, mine_v, rnd_v, acc, rowbuf,
             posb):
    c = lax.axis_index("c")
    s = lax.axis_index("s")
    w = s * 2 + c
    lanes = lax.iota(jnp.int32, 16)
    zf = jnp.zeros((16,), jnp.float32)

    pltpu.sync_copy(idx_hbm, rnd_v)

    def init_acc(i, _):
        for a in range(8):
            for b in range(8):
                acc[a, b, pl.ds(i * 16, 16)] = zf
        return 0

    lax.fori_loop(0, T // 16, init_acc, 0)

    my_cnt = jnp.where(w < HI, CPW + 1, CPW).astype(jnp.int32)
    start = (w * CPW + jnp.minimum(w, HI)).astype(jnp.int32)
    lo = start * T
    hi = (start + my_cnt) * T

    def p1_body(i4, moff):
        offs = moff
        for u in range(4):
            i = i4 * 4 + u
            v = rnd_v[pl.ds(i * 16, 16)]
            m = (v >= lo) & (v < hi)
            psum = plsc.cumsum(jnp.where(m, 1, 0))
            tgt = jnp.maximum(offs + psum - 1, 0)
            pos = i * 16 + lanes
            e = (pos >> 1) | ((pos & 1) << 13) | ((v - lo) << 14)
            plsc.store_scatter(mine_v, [tgt], e, mask=m)
            offs = offs + psum[15]
        return offs

    mcnt = lax.fori_loop(0, N // 64, p1_body, jnp.int32(0))
    n_mv = (mcnt + 15) >> 4

    def round_body(r, _):
        g = start + r

        @pl.when(r < my_cnt)
        def _():
            base = g * T
            rlo = r * T

            def comp_body(i, roff):
                e = mine_v[pl.ds(i * 16, 16)]
                dreg = lax.shift_right_logical(e, 14)
                m = ((dreg >= rlo) & (dreg < rlo + T)
                     & ((i * 16 + lanes) < mcnt))
                psum = plsc.cumsum(jnp.where(m, 1, 0))
                tgt = jnp.maximum(roff + psum - 1, 0)
                re = (e & 0x3FFF) | ((dreg - rlo) << 14)
                plsc.store_scatter(rnd_v, [tgt], re, mask=m)
                return roff + psum[15]

            rcnt = lax.fori_loop(0, n_mv, comp_body, jnp.int32(0))
            nb = (rcnt + (B - 1)) // B

            def batch_body(b, _):
                b0 = b * B
                for q in range(B // 16):
                    e = rnd_v[pl.ds(b0 + q * 16, 16)]
                    posb[pl.ds(q * 16, 16)] = e & 0x1FFF
                pltpu.sync_copy(x_hbm.at[posb], rowbuf.at[:, pl.ds(0, 128)])
                for jg in range(B // 16):
                    jv = jg * 16 + lanes
                    e_vec = rnd_v[pl.ds(b0 + jg * 16, 16)]
                    dv = lax.shift_right_logical(e_vec, 14)
                    colb = (lax.shift_right_logical(e_vec, 13) & 1) * 64
                    valid = (b0 + jg * 16 + lanes) < rcnt
                    for k in range(F):
                        av = jnp.full((16,), k >> 3, jnp.int32)
                        bv = jnp.full((16,), k & 7, jnp.int32)
                        val = plsc.load_gather(rowbuf, [jv, colb + k])
                        plsc.addupdate_scatter(acc, [av, bv, dv], val,
                                               mask=valid)
                return 0

            lax.fori_loop(0, nb, batch_body, 0)

            @pl.when(g != NCH - 1)
            def _():
                for a in range(8):
                    pltpu.sync_copy(
                        acc.at[a],
                        out_hbm.at[pl.ds(a * 8, 8), pl.ds(base, T)])

            @pl.when(g == NCH - 1)
            def _():
                for a in range(8):
                    pltpu.sync_copy(
                        acc.at[a, slice(None), pl.ds(0, LAST_ROWS)],
                        out_hbm.at[pl.ds(a * 8, 8), pl.ds(base, LAST_ROWS)])

            def zero_body(b, _):
                b0 = b * B
                for jg in range(B // 16):
                    e_vec = rnd_v[pl.ds(b0 + jg * 16, 16)]
                    dv = lax.shift_right_logical(e_vec, 14)
                    valid = (b0 + jg * 16 + lanes) < rcnt
                    for k in range(F):
                        av = jnp.full((16,), k >> 3, jnp.int32)
                        bv = jnp.full((16,), k & 7, jnp.int32)
                        plsc.store_scatter(acc, [av, bv, dv], zf, mask=valid)
                return 0

            lax.fori_loop(0, nb, zero_body, 0)

        return 0

    lax.fori_loop(0, CPW + 1, round_body, 0)


@jax.jit
def _impl(x2, idx32):
    kern = pl.kernel(
        _sc_body,
        out_type=jax.ShapeDtypeStruct((F, M), jnp.float32),
        mesh=plsc.VectorSubcoreMesh(core_axis_name="c", subcore_axis_name="s"),
        compiler_params=pltpu.CompilerParams(needs_layout_passes=False),
        scratch_types=[
            pltpu.VMEM((N,), jnp.int32),
            pltpu.VMEM((N,), jnp.int32),
            pltpu.VMEM((8, 8, T), jnp.float32),
            pltpu.VMEM((B, 129), jnp.float32),
            pltpu.VMEM((B,), jnp.int32),
        ],
    )
    return kern(x2, idx32)


def kernel(x_data, scatter_idcs, protoshape):
    idx32 = scatter_idcs[:, 0].astype(jnp.int32)
    x2 = x_data.reshape(N // 2, 128)
    return _impl(x2, idx32).T


# speedup vs baseline: 6.0826x; 1.0481x over previous
import jax
import jax.numpy as jnp
from jax import lax
from jax.experimental import pallas as pl
from jax.experimental.pallas import tpu as pltpu
from jax.experimental.pallas import tpu_sc as plsc

N = 16384
F = 64
M = 1_000_000
MP = 1_000_064
T = 1024
NCH = (MP + T - 1) // T
LAST_ROWS = MP - (NCH - 1) * T
W = 32
B = 32
CPW = NCH // W
HI = NCH - CPW * W


def _sc_body(x_hbm, idx_hbm, out_hbm, mine_v, rnd_v, acc, rowbuf,
             posb, sem):
    c = lax.axis_index("c")
    s = lax.axis_index("s")
    w = s * 2 + c
    lanes = lax.iota(jnp.int32, 16)
    zf = jnp.zeros((16,), jnp.float32)

    pltpu.sync_copy(idx_hbm, rnd_v)

    def init_acc(i, _):
        for a in range(8):
            for b in range(8):
                acc[a, b, pl.ds(i * 16, 16)] = zf
        return 0

    lax.fori_loop(0, T // 16, init_acc, 0)

    my_cnt = jnp.where(w < HI, CPW + 1, CPW).astype(jnp.int32)
    start = (w * CPW + jnp.minimum(w, HI)).astype(jnp.int32)
    lo = start * T
    hi = (start + my_cnt) * T

    def p1_body(i4, moff):
        offs = moff
        for u in range(4):
            i = i4 * 4 + u
            v = rnd_v[pl.ds(i * 16, 16)]
            m = (v >= lo) & (v < hi)
            psum = plsc.cumsum(jnp.where(m, 1, 0))
            tgt = jnp.maximum(offs + psum - 1, 0)
            pos = i * 16 + lanes
            e = (pos >> 1) | ((pos & 1) << 13) | ((v - lo) << 14)
            plsc.store_scatter(mine_v, [tgt], e, mask=m)
            offs = offs + psum[15]
        return offs

    mcnt = lax.fori_loop(0, N // 64, p1_body, jnp.int32(0))
    n_mv = (mcnt + 15) >> 4

    def round_body(r, _):
        g = start + r

        @pl.when(r < my_cnt)
        def _():
            base = g * T
            rlo = r * T

            def comp_body(i, roff):
                e = mine_v[pl.ds(i * 16, 16)]
                dreg = lax.shift_right_logical(e, 14)
                m = ((dreg >= rlo) & (dreg < rlo + T)
                     & ((i * 16 + lanes) < mcnt))
                psum = plsc.cumsum(jnp.where(m, 1, 0))
                tgt = jnp.maximum(roff + psum - 1, 0)
                re = (e & 0x3FFF) | ((dreg - rlo) << 14)
                plsc.store_scatter(rnd_v, [tgt], re, mask=m)
                return roff + psum[15]

            rcnt = lax.fori_loop(0, n_mv, comp_body, jnp.int32(0))
            nb = (rcnt + (B - 1)) // B

            def batch_body(b, _):
                b0 = b * B
                for q in range(B // 16):
                    e = rnd_v[pl.ds(b0 + q * 16, 16)]
                    posb[pl.ds(q * 16, 16)] = e & 0x1FFF
                pltpu.sync_copy(x_hbm.at[posb], rowbuf.at[:, pl.ds(0, 128)])
                for jg in range(B // 16):
                    jv = jg * 16 + lanes
                    e_vec = rnd_v[pl.ds(b0 + jg * 16, 16)]
                    dv = lax.shift_right_logical(e_vec, 14)
                    colb = (lax.shift_right_logical(e_vec, 13) & 1) * 64
                    valid = (b0 + jg * 16 + lanes) < rcnt
                    for k in range(F):
                        av = jnp.full((16,), k >> 3, jnp.int32)
                        bv = jnp.full((16,), k & 7, jnp.int32)
                        val = plsc.load_gather(rowbuf, [jv, colb + k])
                        plsc.addupdate_scatter(acc, [av, bv, dv], val,
                                               mask=valid)
                return 0

            lax.fori_loop(0, nb, batch_body, 0)

            @pl.when(g != NCH - 1)
            def _():
                descs = [
                    pltpu.async_copy(
                        acc.at[a],
                        out_hbm.at[pl.ds(a * 8, 8), pl.ds(base, T)], sem)
                    for a in range(8)]
                for d_ in descs:
                    d_.wait()

            @pl.when(g == NCH - 1)
            def _():
                descs = [
                    pltpu.async_copy(
                        acc.at[a, slice(None), pl.ds(0, LAST_ROWS)],
                        out_hbm.at[pl.ds(a * 8, 8), pl.ds(base, LAST_ROWS)],
                        sem)
                    for a in range(8)]
                for d_ in descs:
                    d_.wait()

            def zero_body(b, _):
                b0 = b * B
                for jg in range(B // 16):
                    e_vec = rnd_v[pl.ds(b0 + jg * 16, 16)]
                    dv = lax.shift_right_logical(e_vec, 14)
                    valid = (b0 + jg * 16 + lanes) < rcnt
                    for k in range(F):
                        av = jnp.full((16,), k >> 3, jnp.int32)
                        bv = jnp.full((16,), k & 7, jnp.int32)
                        plsc.store_scatter(acc, [av, bv, dv], zf, mask=valid)
                return 0

            lax.fori_loop(0, nb, zero_body, 0)

        return 0

    lax.fori_loop(0, CPW + 1, round_body, 0)


@jax.jit
def _impl(x2, idx32):
    kern = pl.kernel(
        _sc_body,
        out_type=jax.ShapeDtypeStruct((F, M), jnp.float32),
        mesh=plsc.VectorSubcoreMesh(core_axis_name="c", subcore_axis_name="s"),
        compiler_params=pltpu.CompilerParams(needs_layout_passes=False),
        scratch_types=[
            pltpu.VMEM((N,), jnp.int32),
            pltpu.VMEM((N,), jnp.int32),
            pltpu.VMEM((8, 8, T), jnp.float32),
            pltpu.VMEM((B, 129), jnp.float32),
            pltpu.VMEM((B,), jnp.int32),
            pltpu.SemaphoreType.DMA,
        ],
    )
    return kern(x2, idx32)


def kernel(x_data, scatter_idcs, protoshape):
    idx32 = scatter_idcs[:, 0].astype(jnp.int32)
    x2 = x_data.reshape(N // 2, 128)
    return _impl(x2, idx32).T
